# baseline (speedup 1.0000x reference)
import functools

import jax
import jax.numpy as jnp
from jax import lax
from jax.experimental import pallas as pl
from jax.experimental.pallas import tpu as pltpu
from jax.experimental.pallas import tpu_sc as plsc

N = 10000
D = 128
K = 10
E = 320000
NP = 10112
NSUB = 16
NC = 2
NT = NC * NSUB
CH = 128
CPT = (-(-E // (NT * CH)) + 7) // 8 * 8
EP = NT * CPT * CH
RPT = NP // NSUB

_mesh = plsc.VectorSubcoreMesh(core_axis_name="c", subcore_axis_name="s")


def _gelu_exact(x):
    return 0.5 * x * (1.0 + lax.erf(x * 0.7071067811865476))


def _layer_norm(x, g, b):
    m = jnp.mean(x, axis=-1, keepdims=True)
    v = jnp.mean((x - m) ** 2, axis=-1, keepdims=True)
    return (x - m) / jnp.sqrt(v + 1e-5) * g + b


def _mlp_body(x_ref, w1, b1, g1, be1, w2, b2, g2, be2, w3, b3, o_ref):
    h = jnp.dot(x_ref[...], w1[...], preferred_element_type=jnp.float32) + b1[...]
    h = _gelu_exact(h)
    h = _layer_norm(h, g1[...], be1[...])
    h = jnp.dot(h, w2[...], preferred_element_type=jnp.float32) + b2[...]
    h = _gelu_exact(h)
    h = _layer_norm(h, g2[...], be2[...])
    o_ref[...] = jnp.dot(h, w3[...], preferred_element_type=jnp.float32) + b3[...]


def _mlp(xp, W1, b1, g1, be1, W2, b2, g2, be2, W3, b3):
    BR = 1264
    full2 = lambda s: pl.BlockSpec(s, lambda i: (0,) * len(s))
    return pl.pallas_call(
        _mlp_body,
        grid=(NP // BR,),
        in_specs=[pl.BlockSpec((BR, D), lambda i: (i, 0)),
                  full2((D, D)), full2((D,)), full2((D,)), full2((D,)),
                  full2((D, D)), full2((D,)), full2((D,)), full2((D,)),
                  full2((D, D)), full2((D,))],
        out_specs=pl.BlockSpec((BR, D), lambda i: (i, 0)),
        out_shape=jax.ShapeDtypeStruct((NP, D), jnp.float32),
    )(xp, W1, b1, g1, be1, W2, b2, g2, be2, W3, b3)


@functools.partial(
    pl.kernel,
    out_type=jax.ShapeDtypeStruct((NC, NP, D), jnp.float32),
    mesh=_mesh,
    scratch_types=[pltpu.VMEM((CPT, CH), jnp.int32),
                   pltpu.VMEM((CH // 2,), jnp.int32),
                   pltpu.VMEM((CH // 2,), jnp.int32),
                   pltpu.VMEM((CH // 2,), jnp.int32),
                   pltpu.VMEM((CH // 2,), jnp.int32),
                   pltpu.VMEM((CH, D), jnp.float32),
                   pltpu.VMEM((CH, D), jnp.float32),
                   pltpu.VMEM_SHARED((NP, D), jnp.float32),
                   pltpu.SemaphoreType.DMA,
                   pltpu.SemaphoreType.DMA,
                   pltpu.SemaphoreType.DMA])
def _hop_kernel(v_hbm, zeros_hbm, rowp_hbm, colp_hbm, out_hbm,
                rows_i, cbuf0a, cbuf0b, cbuf1a, cbuf1b, gbuf0, gbuf1, acc,
                sem0, sem1, semS):
    c = lax.axis_index("c")
    s = lax.axis_index("s")
    wid = c * NSUB + s
    pltpu.sync_copy(rowp_hbm.at[pl.ds(wid * CPT, CPT)], rows_i)
    stripe = pl.ds(s * RPT, RPT)

    @pl.when(c == 0)
    def _():
        pltpu.sync_copy(v_hbm.at[stripe], acc.at[stripe])

    @pl.when(c == 1)
    def _():
        pltpu.sync_copy(zeros_hbm.at[stripe], acc.at[stripe])

    plsc.subcore_barrier()

    SPL = 4
    SW = CH // SPL
    HW = CH // 2

    def fire(j, ca, cb, gbuf, sem):
        pltpu.async_copy(colp_hbm.at[wid * CPT + j, pl.ds(0, HW)], ca, sem)
        pltpu.async_copy(colp_hbm.at[wid * CPT + j, pl.ds(HW, HW)], cb, sem)
        for p in range(SPL):
            pltpu.async_copy(v_hbm.at[rows_i.at[j, pl.ds(p * SW, SW)]],
                             gbuf.at[pl.ds(p * SW, SW)], sem)

    def drain(j, ca, cb, gbuf, sem):
        pltpu.make_async_copy(colp_hbm.at[wid * CPT + j, pl.ds(0, HW)],
                              ca, sem).wait()
        pltpu.make_async_copy(colp_hbm.at[wid * CPT + j, pl.ds(HW, HW)],
                              cb, sem).wait()
        for p in range(SPL):
            pltpu.make_async_copy(v_hbm.at[rows_i.at[j, pl.ds(p * SW, SW)]],
                                  gbuf.at[pl.ds(p * SW, SW)], sem).wait()
        pltpu.async_copy(gbuf.at[pl.ds(0, HW)], acc.at[ca], semS, add=True)
        pltpu.sync_copy(gbuf.at[pl.ds(HW, HW)], acc.at[cb], add=True)
        pltpu.make_async_copy(gbuf.at[pl.ds(0, HW)], acc.at[ca], semS).wait()

    fire(0, cbuf0a, cbuf0b, gbuf0, sem0)

    @pl.loop(0, CPT - 2, step=2)
    def _(j):
        fire(j + 1, cbuf1a, cbuf1b, gbuf1, sem1)
        drain(j, cbuf0a, cbuf0b, gbuf0, sem0)
        fire(j + 2, cbuf0a, cbuf0b, gbuf0, sem0)
        drain(j + 1, cbuf1a, cbuf1b, gbuf1, sem1)

    fire(CPT - 1, cbuf1a, cbuf1b, gbuf1, sem1)
    drain(CPT - 2, cbuf0a, cbuf0b, gbuf0, sem0)
    drain(CPT - 1, cbuf1a, cbuf1b, gbuf1, sem1)

    plsc.subcore_barrier()
    pltpu.sync_copy(acc.at[stripe], out_hbm.at[c, stripe])


def _prep_body(h_ref, d_ref, temp_ref, v0_ref, hv_ref, inv_ref, sq_ref):
    deg = d_ref[0, :, 0:1] + d_ref[1, :, 0:1]
    inv = 1.0 / deg
    dis = lax.rsqrt(deg)
    v0 = h_ref[...] * dis
    v0_ref[...] = v0
    hv_ref[...] = temp_ref[0] * v0
    inv_ref[...] = jnp.broadcast_to(inv, v0.shape)
    sq_ref[...] = jnp.broadcast_to(jnp.sqrt(deg), v0.shape)


def _prep(h, degp, temp):
    BR = 1264
    o = jax.ShapeDtypeStruct((NP, D), jnp.float32)
    return pl.pallas_call(
        _prep_body,
        grid=(NP // BR,),
        in_specs=[pl.BlockSpec((BR, D), lambda i: (i, 0)),
                  pl.BlockSpec((NC, BR, D), lambda i: (0, i, 0)),
                  pl.BlockSpec(memory_space=pltpu.SMEM)],
        out_specs=[pl.BlockSpec((BR, D), lambda i: (i, 0))] * 4,
        out_shape=[o, o, o, o],
    )(h, degp, temp)


def _combine_body(a_ref, inv_ref, hv_ref, tk_ref, v_ref, hvo_ref):
    vn = (a_ref[0] + a_ref[1]) * inv_ref[...]
    v_ref[...] = vn
    hvo_ref[...] = hv_ref[...] + tk_ref[0] * vn


def _combine(a, invdeg, hv, tk):
    BR = 1264
    o = jax.ShapeDtypeStruct((NP, D), jnp.float32)
    return pl.pallas_call(
        _combine_body,
        grid=(NP // BR,),
        in_specs=[pl.BlockSpec((NC, BR, D), lambda i: (0, i, 0)),
                  pl.BlockSpec((BR, D), lambda i: (i, 0)),
                  pl.BlockSpec((BR, D), lambda i: (i, 0)),
                  pl.BlockSpec(memory_space=pltpu.SMEM)],
        out_specs=[pl.BlockSpec((BR, D), lambda i: (i, 0))] * 2,
        out_shape=[o, o],
    )(a, invdeg, hv, tk)


def _final_body(a_ref, inv_ref, hv_ref, sq_ref, tk_ref, o_ref):
    vn = (a_ref[0] + a_ref[1]) * inv_ref[...]
    o_ref[...] = (hv_ref[...] + tk_ref[0] * vn) * sq_ref[...]


def _final(a, invdeg, hv, sqdeg, tk):
    BR = 2000
    return pl.pallas_call(
        _final_body,
        grid=(N // BR,),
        in_specs=[pl.BlockSpec((NC, BR, D), lambda i: (0, i, 0)),
                  pl.BlockSpec((BR, D), lambda i: (i, 0)),
                  pl.BlockSpec((BR, D), lambda i: (i, 0)),
                  pl.BlockSpec((BR, D), lambda i: (i, 0)),
                  pl.BlockSpec(memory_space=pltpu.SMEM)],
        out_specs=pl.BlockSpec((BR, D), lambda i: (i, 0)),
        out_shape=jax.ShapeDtypeStruct((N, D), jnp.float32),
    )(a, invdeg, hv, sqdeg, tk)


def kernel(x, edge_index, W1, b1, g1, be1, W2, b2, g2, be2, W3, b3, temp):
    xp = jnp.pad(x, ((0, NP - N), (0, 0)))
    pad = EP - E
    rowp = jnp.concatenate([edge_index[0], jnp.zeros((pad,), jnp.int32)])
    colp = jnp.concatenate([edge_index[1], jnp.full((pad,), N, jnp.int32)])
    rowp = rowp.reshape(NT * CPT, CH)
    colp = colp.reshape(NT * CPT, CH)

    zerosD = jnp.zeros((NP, D), jnp.float32)
    onesD = jnp.ones((NP, D), jnp.float32)

    h = _mlp(xp, W1, b1, g1, be1, W2, b2, g2, be2, W3, b3)
    dega = _hop_kernel(onesD, zerosD, rowp, colp)
    v, hv, invdeg, sqdeg = _prep(h, dega, temp)

    for k in range(1, K):
        a = _hop_kernel(v, zerosD, rowp, colp)
        v, hv = _combine(a, invdeg, hv, temp[k:k + 1])
    a = _hop_kernel(v, zerosD, rowp, colp)
    return _final(a, invdeg, hv, sqdeg, temp[K:K + 1])


# speedup vs baseline: 1.0785x; 1.0785x over previous
import functools

import jax
import jax.numpy as jnp
from jax import lax
from jax.experimental import pallas as pl
from jax.experimental.pallas import tpu as pltpu
from jax.experimental.pallas import tpu_sc as plsc

N = 10000
D = 128
K = 10
E = 320000
NP = 10112
NSUB = 16
NC = 2
NT = NC * NSUB
CH = 128
CPT = (-(-E // (NT * CH)) + 7) // 8 * 8
EP = NT * CPT * CH
RPT = NP // NSUB

_mesh = plsc.VectorSubcoreMesh(core_axis_name="c", subcore_axis_name="s")


def _gelu_exact(x):
    return 0.5 * x * (1.0 + lax.erf(x * 0.7071067811865476))


def _layer_norm(x, g, b):
    m = jnp.mean(x, axis=-1, keepdims=True)
    v = jnp.mean((x - m) ** 2, axis=-1, keepdims=True)
    return (x - m) / jnp.sqrt(v + 1e-5) * g + b


def _mlp_body(x_ref, w1, b1, g1, be1, w2, b2, g2, be2, w3, b3, o_ref):
    h = jnp.dot(x_ref[...], w1[...], preferred_element_type=jnp.float32) + b1[...]
    h = _gelu_exact(h)
    h = _layer_norm(h, g1[...], be1[...])
    h = jnp.dot(h, w2[...], preferred_element_type=jnp.float32) + b2[...]
    h = _gelu_exact(h)
    h = _layer_norm(h, g2[...], be2[...])
    o_ref[...] = jnp.dot(h, w3[...], preferred_element_type=jnp.float32) + b3[...]


def _mlp(xp, W1, b1, g1, be1, W2, b2, g2, be2, W3, b3):
    BR = 1264
    full2 = lambda s: pl.BlockSpec(s, lambda i: (0,) * len(s))
    return pl.pallas_call(
        _mlp_body,
        grid=(NP // BR,),
        in_specs=[pl.BlockSpec((BR, D), lambda i: (i, 0)),
                  full2((D, D)), full2((D,)), full2((D,)), full2((D,)),
                  full2((D, D)), full2((D,)), full2((D,)), full2((D,)),
                  full2((D, D)), full2((D,))],
        out_specs=pl.BlockSpec((BR, D), lambda i: (i, 0)),
        out_shape=jax.ShapeDtypeStruct((NP, D), jnp.float32),
    )(xp, W1, b1, g1, be1, W2, b2, g2, be2, W3, b3)


@functools.partial(
    pl.kernel,
    out_type=jax.ShapeDtypeStruct((NC, NP, D), jnp.float32),
    mesh=_mesh,
    scratch_types=[pltpu.VMEM((CPT, CH), jnp.int32),
                   pltpu.VMEM((CH // 2,), jnp.int32),
                   pltpu.VMEM((CH // 2,), jnp.int32),
                   pltpu.VMEM((CH // 2,), jnp.int32),
                   pltpu.VMEM((CH // 2,), jnp.int32),
                   pltpu.VMEM((CH, D), jnp.float32),
                   pltpu.VMEM((CH, D), jnp.float32),
                   pltpu.VMEM_SHARED((NP, D), jnp.float32),
                   pltpu.SemaphoreType.DMA,
                   pltpu.SemaphoreType.DMA,
                   pltpu.SemaphoreType.DMA])
def _hop_kernel(v_hbm, zeros_hbm, rowp_hbm, colp_hbm, out_hbm,
                rows_i, cbuf0a, cbuf0b, cbuf1a, cbuf1b, gbuf0, gbuf1, acc,
                sem0, sem1, semS):
    c = lax.axis_index("c")
    s = lax.axis_index("s")
    wid = c * NSUB + s
    pltpu.sync_copy(rowp_hbm.at[pl.ds(wid * CPT, CPT)], rows_i)
    stripe = pl.ds(s * RPT, RPT)

    @pl.when(c == 0)
    def _():
        pltpu.sync_copy(v_hbm.at[stripe], acc.at[stripe])

    @pl.when(c == 1)
    def _():
        pltpu.sync_copy(zeros_hbm.at[stripe], acc.at[stripe])

    plsc.subcore_barrier()

    SPL = 4
    SW = CH // SPL
    HW = CH // 2

    def fire(j, ca, cb, gbuf, sem):
        pltpu.async_copy(colp_hbm.at[wid * CPT + j, pl.ds(0, HW)], ca, sem)
        pltpu.async_copy(colp_hbm.at[wid * CPT + j, pl.ds(HW, HW)], cb, sem)
        for p in range(SPL):
            pltpu.async_copy(v_hbm.at[rows_i.at[j, pl.ds(p * SW, SW)]],
                             gbuf.at[pl.ds(p * SW, SW)], sem)

    def drain(j, ca, cb, gbuf, sem):
        pltpu.make_async_copy(colp_hbm.at[wid * CPT + j, pl.ds(0, HW)],
                              ca, sem).wait()
        pltpu.make_async_copy(colp_hbm.at[wid * CPT + j, pl.ds(HW, HW)],
                              cb, sem).wait()
        for p in range(SPL):
            pltpu.make_async_copy(v_hbm.at[rows_i.at[j, pl.ds(p * SW, SW)]],
                                  gbuf.at[pl.ds(p * SW, SW)], sem).wait()
        pltpu.async_copy(gbuf.at[pl.ds(0, HW)], acc.at[ca], semS, add=True)
        pltpu.sync_copy(gbuf.at[pl.ds(HW, HW)], acc.at[cb], add=True)
        pltpu.make_async_copy(gbuf.at[pl.ds(0, HW)], acc.at[ca], semS).wait()

    fire(0, cbuf0a, cbuf0b, gbuf0, sem0)

    @pl.loop(0, CPT - 2, step=2)
    def _(j):
        fire(j + 1, cbuf1a, cbuf1b, gbuf1, sem1)
        drain(j, cbuf0a, cbuf0b, gbuf0, sem0)
        fire(j + 2, cbuf0a, cbuf0b, gbuf0, sem0)
        drain(j + 1, cbuf1a, cbuf1b, gbuf1, sem1)

    fire(CPT - 1, cbuf1a, cbuf1b, gbuf1, sem1)
    drain(CPT - 2, cbuf0a, cbuf0b, gbuf0, sem0)
    drain(CPT - 1, cbuf1a, cbuf1b, gbuf1, sem1)

    plsc.subcore_barrier()
    pltpu.sync_copy(acc.at[stripe], out_hbm.at[c, stripe])


@functools.partial(
    pl.kernel,
    out_type=jax.ShapeDtypeStruct((NC, NP, D), jnp.float32),
    mesh=_mesh,
    scratch_types=[pltpu.VMEM((CH, D), jnp.float32),
                   pltpu.VMEM((CH,), jnp.int32),
                   pltpu.VMEM((CH,), jnp.int32),
                   pltpu.VMEM_SHARED((NP, D), jnp.float32),
                   pltpu.SemaphoreType.DMA,
                   pltpu.SemaphoreType.DMA])
def _deg_kernel(ones_hbm, zeros_hbm, colp_hbm, out_hbm,
                ones_v, cbuf0, cbuf1, acc, sem0, sem1):
    c = lax.axis_index("c")
    s = lax.axis_index("s")
    wid = c * NSUB + s
    pltpu.sync_copy(ones_hbm.at[pl.ds(0, CH)], ones_v)
    stripe = pl.ds(s * RPT, RPT)

    @pl.when(c == 0)
    def _():
        pltpu.sync_copy(ones_hbm.at[stripe], acc.at[stripe])

    @pl.when(c == 1)
    def _():
        pltpu.sync_copy(zeros_hbm.at[stripe], acc.at[stripe])

    plsc.subcore_barrier()

    def fire(j, cbuf, sem):
        pltpu.async_copy(colp_hbm.at[wid * CPT + j], cbuf, sem)

    def drain(j, cbuf, sem):
        pltpu.make_async_copy(colp_hbm.at[wid * CPT + j], cbuf, sem).wait()
        pltpu.sync_copy(ones_v, acc.at[cbuf], add=True)

    fire(0, cbuf0, sem0)

    @pl.loop(0, CPT - 2, step=2)
    def _(j):
        fire(j + 1, cbuf1, sem1)
        drain(j, cbuf0, sem0)
        fire(j + 2, cbuf0, sem0)
        drain(j + 1, cbuf1, sem1)

    fire(CPT - 1, cbuf1, sem1)
    drain(CPT - 2, cbuf0, sem0)
    drain(CPT - 1, cbuf1, sem1)

    plsc.subcore_barrier()
    pltpu.sync_copy(acc.at[stripe], out_hbm.at[c, stripe])


def _prep_body(h_ref, d_ref, temp_ref, v0_ref, hv_ref, inv_ref, sq_ref):
    deg = d_ref[0, :, 0:1] + d_ref[1, :, 0:1]
    inv = 1.0 / deg
    dis = lax.rsqrt(deg)
    v0 = h_ref[...] * dis
    v0_ref[...] = v0
    hv_ref[...] = temp_ref[0] * v0
    inv_ref[...] = jnp.broadcast_to(inv, v0.shape)
    sq_ref[...] = jnp.broadcast_to(jnp.sqrt(deg), v0.shape)


def _prep(h, degp, temp):
    BR = 1264
    o = jax.ShapeDtypeStruct((NP, D), jnp.float32)
    return pl.pallas_call(
        _prep_body,
        grid=(NP // BR,),
        in_specs=[pl.BlockSpec((BR, D), lambda i: (i, 0)),
                  pl.BlockSpec((NC, BR, D), lambda i: (0, i, 0)),
                  pl.BlockSpec(memory_space=pltpu.SMEM)],
        out_specs=[pl.BlockSpec((BR, D), lambda i: (i, 0))] * 4,
        out_shape=[o, o, o, o],
    )(h, degp, temp)


def _combine_body(a_ref, inv_ref, hv_ref, tk_ref, v_ref, hvo_ref):
    vn = (a_ref[0] + a_ref[1]) * inv_ref[...]
    v_ref[...] = vn
    hvo_ref[...] = hv_ref[...] + tk_ref[0] * vn


def _combine(a, invdeg, hv, tk):
    BR = 1264
    o = jax.ShapeDtypeStruct((NP, D), jnp.float32)
    return pl.pallas_call(
        _combine_body,
        grid=(NP // BR,),
        in_specs=[pl.BlockSpec((NC, BR, D), lambda i: (0, i, 0)),
                  pl.BlockSpec((BR, D), lambda i: (i, 0)),
                  pl.BlockSpec((BR, D), lambda i: (i, 0)),
                  pl.BlockSpec(memory_space=pltpu.SMEM)],
        out_specs=[pl.BlockSpec((BR, D), lambda i: (i, 0))] * 2,
        out_shape=[o, o],
    )(a, invdeg, hv, tk)


def _final_body(a_ref, inv_ref, hv_ref, sq_ref, tk_ref, o_ref):
    vn = (a_ref[0] + a_ref[1]) * inv_ref[...]
    o_ref[...] = (hv_ref[...] + tk_ref[0] * vn) * sq_ref[...]


def _final(a, invdeg, hv, sqdeg, tk):
    BR = 2000
    return pl.pallas_call(
        _final_body,
        grid=(N // BR,),
        in_specs=[pl.BlockSpec((NC, BR, D), lambda i: (0, i, 0)),
                  pl.BlockSpec((BR, D), lambda i: (i, 0)),
                  pl.BlockSpec((BR, D), lambda i: (i, 0)),
                  pl.BlockSpec((BR, D), lambda i: (i, 0)),
                  pl.BlockSpec(memory_space=pltpu.SMEM)],
        out_specs=pl.BlockSpec((BR, D), lambda i: (i, 0)),
        out_shape=jax.ShapeDtypeStruct((N, D), jnp.float32),
    )(a, invdeg, hv, sqdeg, tk)


def kernel(x, edge_index, W1, b1, g1, be1, W2, b2, g2, be2, W3, b3, temp):
    xp = jnp.pad(x, ((0, NP - N), (0, 0)))
    pad = EP - E
    rowp = jnp.concatenate([edge_index[0], jnp.zeros((pad,), jnp.int32)])
    colp = jnp.concatenate([edge_index[1], jnp.full((pad,), N, jnp.int32)])
    rowp = rowp.reshape(NT * CPT, CH)
    colp = colp.reshape(NT * CPT, CH)

    zerosD = jnp.zeros((NP, D), jnp.float32)
    onesD = jnp.ones((NP, D), jnp.float32)

    h = _mlp(xp, W1, b1, g1, be1, W2, b2, g2, be2, W3, b3)
    dega = _deg_kernel(onesD, zerosD, colp)
    v, hv, invdeg, sqdeg = _prep(h, dega, temp)

    for k in range(1, K):
        a = _hop_kernel(v, zerosD, rowp, colp)
        v, hv = _combine(a, invdeg, hv, temp[k:k + 1])
    a = _hop_kernel(v, zerosD, rowp, colp)
    return _final(a, invdeg, hv, sqdeg, temp[K:K + 1])
